# baseline (speedup 1.0000x reference)
import jax
import jax.numpy as jnp
from jax.experimental import pallas as pl
from jax.experimental.pallas import tpu as pltpu


def _linear_t_kernel(x_ref, w_ref, b_ref, o_ref, wb_ref, bt_ref):
    @pl.when(pl.program_id(0) == 0)
    def _():
        wb_ref[...] = w_ref[...].astype(jnp.bfloat16)
        bt_ref[...] = jnp.transpose(b_ref[...], (1, 0))

    xb = x_ref[...].astype(jnp.bfloat16)
    acc = jax.lax.dot_general(
        wb_ref[...], xb, (((1,), (1,)), ((), ())),
        preferred_element_type=jnp.float32)
    o_ref[...] = (acc + bt_ref[...]).astype(o_ref.dtype)


def kernel(x, weight, bias):
    N, dim = x.shape
    n_way = weight.shape[0]
    out_dtype = x.dtype
    esz = jnp.dtype(out_dtype).itemsize

    b2 = bias.reshape(1, n_way).astype(jnp.float32)

    tm = 512
    if N % tm != 0:
        tm = 8 * pl.cdiv(N, 8 * pl.cdiv(N, tm))
    grid_m = pl.cdiv(N, tm)

    cost = pl.CostEstimate(
        flops=2 * N * dim * n_way,
        transcendentals=0,
        bytes_accessed=esz * (N * dim + N * n_way + n_way * dim))

    out_t = pl.pallas_call(
        _linear_t_kernel,
        out_shape=jax.ShapeDtypeStruct((n_way, N), out_dtype),
        grid=(grid_m,),
        in_specs=[
            pl.BlockSpec((tm, dim), lambda i: (i, 0)),
            pl.BlockSpec((n_way, dim), lambda i: (0, 0)),
            pl.BlockSpec((1, n_way), lambda i: (0, 0)),
        ],
        out_specs=pl.BlockSpec((n_way, tm), lambda i: (0, i)),
        scratch_shapes=[
            pltpu.VMEM((n_way, dim), jnp.bfloat16),
            pltpu.VMEM((n_way, 1), jnp.float32),
        ],
        compiler_params=pltpu.CompilerParams(
            dimension_semantics=("arbitrary",),
            vmem_limit_bytes=56 * 1024 * 1024),
        cost_estimate=cost,
    )(x, weight, b2)
    return jnp.transpose(out_t)


# speedup vs baseline: 1.0765x; 1.0765x over previous
import jax
import jax.numpy as jnp
from jax.experimental import pallas as pl
from jax.experimental.pallas import tpu as pltpu


def _linear_t_kernel(x_ref, w_ref, b_ref, o_ref, wb_ref, bt_ref):
    @pl.when(pl.program_id(0) == 0)
    def _():
        wb_ref[...] = w_ref[...].astype(jnp.bfloat16)
        bt_ref[...] = jnp.transpose(b_ref[...], (1, 0))

    xb = x_ref[...].astype(jnp.bfloat16)
    acc = jax.lax.dot_general(
        wb_ref[...], xb, (((1,), (1,)), ((), ())),
        preferred_element_type=jnp.float32)
    o_ref[...] = (acc + bt_ref[...]).astype(o_ref.dtype)


def kernel(x, weight, bias):
    N, dim = x.shape
    n_way = weight.shape[0]
    out_dtype = x.dtype
    esz = jnp.dtype(out_dtype).itemsize

    b2 = bias.reshape(1, n_way).astype(jnp.float32)

    tm = 1024
    if N % tm != 0:
        tm = 8 * pl.cdiv(N, 8 * pl.cdiv(N, tm))
    grid_m = pl.cdiv(N, tm)

    cost = pl.CostEstimate(
        flops=2 * N * dim * n_way,
        transcendentals=0,
        bytes_accessed=esz * (N * dim + N * n_way + n_way * dim))

    out_t = pl.pallas_call(
        _linear_t_kernel,
        out_shape=jax.ShapeDtypeStruct((n_way, N), out_dtype),
        grid=(grid_m,),
        in_specs=[
            pl.BlockSpec((tm, dim), lambda i: (i, 0)),
            pl.BlockSpec((n_way, dim), lambda i: (0, 0)),
            pl.BlockSpec((1, n_way), lambda i: (0, 0)),
        ],
        out_specs=pl.BlockSpec((n_way, tm), lambda i: (0, i)),
        scratch_shapes=[
            pltpu.VMEM((n_way, dim), jnp.bfloat16),
            pltpu.VMEM((n_way, 1), jnp.float32),
        ],
        compiler_params=pltpu.CompilerParams(
            dimension_semantics=("arbitrary",),
            vmem_limit_bytes=56 * 1024 * 1024),
        cost_estimate=cost,
    )(x, weight, b2)
    return jnp.transpose(out_t)
